# baseline (speedup 1.0000x reference)
import functools

import jax
import jax.numpy as jnp
from jax import lax
from jax.experimental import pallas as pl
from jax.experimental.pallas import tpu as pltpu, tpu_sc as plsc

N = 10000
E = 160000
D = 256
HALF = 128
OUT = 128
L = 3
EPS_BN = 1e-5

NC = 2
NS = 16
CHUNK = 128
NCHUNK = 80
EPT = CHUNK * NCHUNK
E_PAD = EPT * NS
ACC_ROWS = 10240
ROWS_PER_TILE_ZERO = ACC_ROWS // NS
ROWS_PER_TILE_OUT = 632
N_OUT = ROWS_PER_TILE_OUT * NS
NBUF = 2
NPHASE = 2
PCHUNK = NCHUNK // NPHASE

_mesh = plsc.VectorSubcoreMesh(
    core_axis_name="c", subcore_axis_name="s", num_cores=NC, num_subcores=NS)


_SC_SCRATCH = [
    pltpu.VMEM((PCHUNK, CHUNK), jnp.int32),
    pltpu.VMEM((PCHUNK, CHUNK), jnp.int32),
    pltpu.VMEM((CHUNK, HALF), jnp.float32),
    pltpu.VMEM((CHUNK, HALF), jnp.float32),
    pltpu.VMEM_SHARED((ACC_ROWS, HALF), jnp.float32),
    pltpu.SemaphoreType.DMA,
    pltpu.SemaphoreType.DMA,
]


def _sc_body(h_hbm, src_hbm, dst_hbm, out_hbm,
             sidx, didx, b0, b1, acc, s0, s1):
    c = lax.axis_index("c")
    s = lax.axis_index("s")
    bufs = (b0, b1)
    sems = (s0, s1)

    @pl.loop(0, CHUNK)
    def _zero_row(r):
        for k in range(HALF // 16):
            b0[r, pl.ds(k * 16, 16)] = jnp.zeros((16,), jnp.float32)

    for k in range(ROWS_PER_TILE_ZERO // CHUNK):
        pltpu.sync_copy(b0, acc.at[pl.ds(s * ROWS_PER_TILE_ZERO + k * CHUNK, CHUNK)])
    plsc.subcore_barrier()

    for p in range(NPHASE):
        pltpu.sync_copy(src_hbm.at[c, s, pl.ds(p * PCHUNK, PCHUNK)], sidx)
        pltpu.sync_copy(dst_hbm.at[s, pl.ds(p * PCHUNK, PCHUNK)], didx)

        for b in range(NBUF):
            pltpu.async_copy(h_hbm.at[sidx.at[b]], bufs[b], sems[b])

        @pl.loop(0, PCHUNK // NBUF - 1)
        def _main(t):
            j0 = t * NBUF
            for b in range(NBUF):
                j = j0 + b
                pltpu.make_async_copy(h_hbm.at[sidx.at[j]], bufs[b], sems[b]).wait()
                pltpu.sync_copy(bufs[b], acc.at[didx.at[j]], add=True)
                pltpu.async_copy(h_hbm.at[sidx.at[j + NBUF]], bufs[b], sems[b])

        for b in range(NBUF):
            j = PCHUNK - NBUF + b
            pltpu.make_async_copy(h_hbm.at[sidx.at[j]], bufs[b], sems[b]).wait()
            pltpu.sync_copy(bufs[b], acc.at[didx.at[j]], add=True)

    plsc.subcore_barrier()
    pltpu.sync_copy(acc.at[pl.ds(s * ROWS_PER_TILE_OUT, ROWS_PER_TILE_OUT)],
                    out_hbm.at[c, pl.ds(s * ROWS_PER_TILE_OUT, ROWS_PER_TILE_OUT)])


_sc_aggregate = pl.kernel(
    _sc_body,
    out_type=jax.ShapeDtypeStruct((NC, N_OUT, HALF), jnp.float32),
    mesh=_mesh,
    scratch_types=_SC_SCRATCH,
)


DEGW = HALF


def _sc_deg_body(dst_hbm, out_hbm, didx, ones, acc, s0):
    c = lax.axis_index("c")
    s = lax.axis_index("s")

    @pl.loop(0, CHUNK)
    def _fill_zero(r):
        for k in range(DEGW // 16):
            ones[r, pl.ds(k * 16, 16)] = jnp.zeros((16,), jnp.float32)

    for k in range(ROWS_PER_TILE_ZERO // CHUNK):
        pltpu.sync_copy(ones, acc.at[pl.ds(s * ROWS_PER_TILE_ZERO + k * CHUNK, CHUNK)])

    @pl.loop(0, CHUNK)
    def _fill_one(r):
        for k in range(DEGW // 16):
            ones[r, pl.ds(k * 16, 16)] = jnp.ones((16,), jnp.float32)

    plsc.subcore_barrier()

    pltpu.sync_copy(dst_hbm.at[s, pl.ds(c * PCHUNK, PCHUNK)], didx)

    @pl.loop(0, PCHUNK)
    def _fire(j):
        pltpu.async_copy(ones, acc.at[didx.at[j]], s0, add=True)

    @pl.loop(0, PCHUNK)
    def _drain(j):
        pltpu.make_async_copy(ones, acc.at[didx.at[j]], s0).wait()

    plsc.subcore_barrier()
    pltpu.sync_copy(acc.at[pl.ds(s * ROWS_PER_TILE_OUT, ROWS_PER_TILE_OUT)],
                    out_hbm.at[c, pl.ds(s * ROWS_PER_TILE_OUT, ROWS_PER_TILE_OUT)])


_sc_degree = pl.kernel(
    _sc_deg_body,
    out_type=jax.ShapeDtypeStruct((NC, N_OUT, DEGW), jnp.float32),
    mesh=_mesh,
    scratch_types=[
        pltpu.VMEM((PCHUNK, CHUNK), jnp.int32),
        pltpu.VMEM((CHUNK, DEGW), jnp.float32),
        pltpu.VMEM_SHARED((ACC_ROWS, DEGW), jnp.float32),
        pltpu.SemaphoreType.DMA,
    ],
)


ROWS = 2000
GRID = N // ROWS
_DOT = dict(preferred_element_type=jnp.float32)


def _store_halves(out_ref, z2):
    out_ref[0] = z2[:, :HALF]
    out_ref[1] = z2[:, HALF:]


def _mlp_tail(i, z1_pre, w2_ref, bb2_ref, z2_ref, st_ref):
    z1 = jnp.maximum(z1_pre, 0.0)
    z2 = jnp.maximum(
        lax.dot_general(z1, w2_ref[...], (((1,), (0,)), ((), ())), **_DOT)
        + bb2_ref[...], 0.0)
    _store_halves(z2_ref, z2)

    @pl.when(i == 0)
    def _():
        st_ref[...] = jnp.zeros_like(st_ref)

    st_ref[0:1, :] = st_ref[0:1, :] + jnp.sum(z2, axis=0, keepdims=True)
    st_ref[1:2, :] = st_ref[1:2, :] + jnp.sum(z2 * z2, axis=0, keepdims=True)


def _mlp_body(h_ref, a_ref, w1_ref, bb1_ref, w2_ref, bb2_ref, z2_ref, st_ref):
    i = pl.program_id(0)
    z = jnp.concatenate([h_ref[0] + a_ref[0], h_ref[1] + a_ref[1]], axis=-1)
    z1_pre = (lax.dot_general(z, w1_ref[...], (((1,), (0,)), ((), ())), **_DOT)
              + bb1_ref[...])
    _mlp_tail(i, z1_pre, w2_ref, bb2_ref, z2_ref, st_ref)


def _bn_affine(st_ref, g_ref, be_ref):
    mean = st_ref[0:1, :] * (1.0 / N)
    ex2 = st_ref[1:2, :] * (1.0 / N)
    var = jnp.maximum(ex2 - mean * mean, 0.0)
    a = g_ref[...] * lax.rsqrt(var + EPS_BN)
    b = be_ref[...] - mean * a
    return a, b


def _mlp_fold_body(z2p_ref, aggz_ref, stp_ref, gp_ref, bep_ref, deg_ref,
                   w1_ref, bb1_ref, w2_ref, bb2_ref, z2_ref, st_ref):
    i = pl.program_id(0)
    a, b = _bn_affine(stp_ref, gp_ref, bep_ref)
    zsum = jnp.concatenate([z2p_ref[0] + aggz_ref[0], z2p_ref[1] + aggz_ref[1]],
                           axis=-1)
    deg = deg_ref[0, :, 0:1] + deg_ref[1, :, 0:1]
    z = a * zsum + (1.0 + deg) * b
    z1_pre = (lax.dot_general(z, w1_ref[...], (((1,), (0,)), ((), ())), **_DOT)
              + bb1_ref[...])
    _mlp_tail(i, z1_pre, w2_ref, bb2_ref, z2_ref, st_ref)


_MLP_OUT = dict(
    out_specs=[
        pl.BlockSpec((NC, ROWS, HALF), lambda i: (0, i, 0)),
        pl.BlockSpec((8, D), lambda i: (0, 0)),
    ],
    out_shape=[
        jax.ShapeDtypeStruct((NC, N, HALF), jnp.float32),
        jax.ShapeDtypeStruct((8, D), jnp.float32),
    ],
    compiler_params=pltpu.CompilerParams(dimension_semantics=("arbitrary",)),
)

_mlp = pl.pallas_call(
    _mlp_body,
    grid=(GRID,),
    in_specs=[
        pl.BlockSpec((NC, ROWS, HALF), lambda i: (0, i, 0)),
        pl.BlockSpec((NC, ROWS, HALF), lambda i: (0, i, 0)),
        pl.BlockSpec((D, D), lambda i: (0, 0)),
        pl.BlockSpec((1, D), lambda i: (0, 0)),
        pl.BlockSpec((D, D), lambda i: (0, 0)),
        pl.BlockSpec((1, D), lambda i: (0, 0)),
    ],
    **_MLP_OUT,
)

_mlp_fold = pl.pallas_call(
    _mlp_fold_body,
    grid=(GRID,),
    in_specs=[
        pl.BlockSpec((NC, ROWS, HALF), lambda i: (0, i, 0)),
        pl.BlockSpec((NC, ROWS, HALF), lambda i: (0, i, 0)),
        pl.BlockSpec((8, D), lambda i: (0, 0)),
        pl.BlockSpec((1, D), lambda i: (0, 0)),
        pl.BlockSpec((1, D), lambda i: (0, 0)),
        pl.BlockSpec((NC, ROWS, DEGW), lambda i: (0, i, 0)),
        pl.BlockSpec((D, D), lambda i: (0, 0)),
        pl.BlockSpec((1, D), lambda i: (0, 0)),
        pl.BlockSpec((D, D), lambda i: (0, 0)),
        pl.BlockSpec((1, D), lambda i: (0, 0)),
    ],
    **_MLP_OUT,
)


def _norm_proj_body(z2_ref, st_ref, g_ref, be_ref, wp_ref, bbp_ref, out_ref):
    a, b = _bn_affine(st_ref, g_ref, be_ref)
    z2 = jnp.concatenate([z2_ref[0], z2_ref[1]], axis=-1)
    hn = a * z2 + b
    out_ref[...] = (
        lax.dot_general(hn, wp_ref[...], (((1,), (0,)), ((), ())), **_DOT)
        + bbp_ref[...])


_norm_proj = pl.pallas_call(
    _norm_proj_body,
    grid=(GRID,),
    in_specs=[
        pl.BlockSpec((NC, ROWS, HALF), lambda i: (0, i, 0)),
        pl.BlockSpec((8, D), lambda i: (0, 0)),
        pl.BlockSpec((1, D), lambda i: (0, 0)),
        pl.BlockSpec((1, D), lambda i: (0, 0)),
        pl.BlockSpec((D, OUT), lambda i: (0, 0)),
        pl.BlockSpec((1, OUT), lambda i: (0, 0)),
    ],
    out_specs=pl.BlockSpec((ROWS, OUT), lambda i: (i, 0)),
    out_shape=jax.ShapeDtypeStruct((N, OUT), jnp.float32),
    compiler_params=pltpu.CompilerParams(
        dimension_semantics=("arbitrary",)),
)


def kernel(x, edge_index, batch, W1, b1, W2, b2, gamma, beta, Wp, bp):
    del batch
    src = edge_index[0]
    dst = edge_index[1]

    npad = E_PAD - E
    pad_ids = jnp.arange(npad, dtype=jnp.int32)
    src_p = jnp.concatenate([src, pad_ids % N])
    dst_p = jnp.concatenate([dst, N + pad_ids % (ACC_ROWS - N)])
    src2 = jnp.stack([src_p, src_p + N]).reshape(NC, NS, NCHUNK, CHUNK)
    dst_r = dst_p.reshape(NS, NCHUNK, CHUNK)

    x2 = jnp.stack([x[:, :HALF], x[:, HALF:]])
    agg = _sc_aggregate(x2.reshape(NC * N, HALF), src2, dst_r)
    deg = _sc_degree(dst_r)
    z2, st = _mlp(x2, agg, W1[0].T, b1[0].reshape(1, D),
                  W2[0].T, b2[0].reshape(1, D))

    for l in range(1, L):
        aggz = _sc_aggregate(z2.reshape(NC * N, HALF), src2, dst_r)
        z2, st = _mlp_fold(z2, aggz, st,
                           gamma[l - 1].reshape(1, D), beta[l - 1].reshape(1, D),
                           deg, W1[l].T, b1[l].reshape(1, D),
                           W2[l].T, b2[l].reshape(1, D))

    return _norm_proj(z2, st, gamma[L - 1].reshape(1, D),
                      beta[L - 1].reshape(1, D), Wp.T, bp.reshape(1, OUT))


# speedup vs baseline: 1.1119x; 1.1119x over previous
import jax
import jax.numpy as jnp
from jax import lax
from jax.experimental import pallas as pl
from jax.experimental.pallas import tpu as pltpu, tpu_sc as plsc

N = 10000
E = 160000
D = 256
HALF = 128
OUT = 128
L = 3
EPS_BN = 1e-5

NC = 2
NS = 16
CHUNK = 128
NCHUNK = 80
EPT = CHUNK * NCHUNK
E_PAD = EPT * NS
ACC_ROWS = 10240
ROWS_PER_TILE_ZERO = ACC_ROWS // NS
ROWS_PER_TILE_OUT = 632
N_OUT = ROWS_PER_TILE_OUT * NS
NBUF = 2
NPHASE = 2
PCHUNK = NCHUNK // NPHASE

_mesh = plsc.VectorSubcoreMesh(
    core_axis_name="c", subcore_axis_name="s", num_cores=NC, num_subcores=NS)

_SC_SCRATCH = [
    pltpu.VMEM((PCHUNK, CHUNK), jnp.int32),
    pltpu.VMEM((PCHUNK, CHUNK), jnp.int32),
    pltpu.VMEM((CHUNK, HALF), jnp.float32),
    pltpu.VMEM((CHUNK, HALF), jnp.float32),
    pltpu.VMEM_SHARED((ACC_ROWS, HALF), jnp.float32),
    pltpu.SemaphoreType.DMA,
    pltpu.SemaphoreType.DMA,
]


def _sc_body(h_hbm, src_hbm, dst_hbm, out_hbm,
             sidx, didx, b0, b1, acc, s0, s1):
    c = lax.axis_index("c")
    s = lax.axis_index("s")
    bufs = (b0, b1)
    sems = (s0, s1)

    @pl.loop(0, CHUNK)
    def _zero_row(r):
        for k in range(HALF // 16):
            b0[r, pl.ds(k * 16, 16)] = jnp.zeros((16,), jnp.float32)

    for k in range(ROWS_PER_TILE_ZERO // CHUNK):
        pltpu.sync_copy(b0, acc.at[pl.ds(s * ROWS_PER_TILE_ZERO + k * CHUNK, CHUNK)])
    plsc.subcore_barrier()

    for p in range(NPHASE):
        pltpu.sync_copy(src_hbm.at[c, s, pl.ds(p * PCHUNK, PCHUNK)], sidx)
        pltpu.sync_copy(dst_hbm.at[s, pl.ds(p * PCHUNK, PCHUNK)], didx)

        for b in range(NBUF):
            pltpu.async_copy(h_hbm.at[sidx.at[b]], bufs[b], sems[b])

        @pl.loop(0, PCHUNK // NBUF - 1)
        def _main(t):
            j0 = t * NBUF
            for b in range(NBUF):
                j = j0 + b
                pltpu.make_async_copy(h_hbm.at[sidx.at[j]], bufs[b], sems[b]).wait()
                pltpu.sync_copy(bufs[b], acc.at[didx.at[j]], add=True)
                pltpu.async_copy(h_hbm.at[sidx.at[j + NBUF]], bufs[b], sems[b])

        for b in range(NBUF):
            j = PCHUNK - NBUF + b
            pltpu.make_async_copy(h_hbm.at[sidx.at[j]], bufs[b], sems[b]).wait()
            pltpu.sync_copy(bufs[b], acc.at[didx.at[j]], add=True)

    plsc.subcore_barrier()
    pltpu.sync_copy(acc.at[pl.ds(s * ROWS_PER_TILE_OUT, ROWS_PER_TILE_OUT)],
                    out_hbm.at[c, pl.ds(s * ROWS_PER_TILE_OUT, ROWS_PER_TILE_OUT)])


_sc_aggregate = pl.kernel(
    _sc_body,
    out_type=jax.ShapeDtypeStruct((NC, N_OUT, HALF), jnp.float32),
    mesh=_mesh,
    scratch_types=_SC_SCRATCH,
)


ROWS = 2000
GRID = N // ROWS
_DOT = dict(preferred_element_type=jnp.float32)


def _mlp_block(h_ref, a_ref, w1_ref, bb1_ref, w2_ref, bb2_ref):
    z = jnp.concatenate([h_ref[0] + a_ref[0], h_ref[1] + a_ref[1]], axis=-1)
    z1 = jnp.maximum(
        lax.dot_general(z, w1_ref[...], (((1,), (0,)), ((), ())), **_DOT)
        + bb1_ref[...], 0.0)
    return jnp.maximum(
        lax.dot_general(z1, w2_ref[...], (((1,), (0,)), ((), ())), **_DOT)
        + bb2_ref[...], 0.0)


def _phase0(b, h_ref, a_ref, w1_ref, bb1_ref, w2_ref, bb2_ref, z2s, st):
    z2 = _mlp_block(h_ref, a_ref, w1_ref, bb1_ref, w2_ref, bb2_ref)
    z2s[pl.ds(b * ROWS, ROWS), :] = z2

    @pl.when(b == 0)
    def _():
        st[...] = jnp.zeros_like(st)

    st[0:1, :] = st[0:1, :] + jnp.sum(z2, axis=0, keepdims=True)
    st[1:2, :] = st[1:2, :] + jnp.sum(z2 * z2, axis=0, keepdims=True)


def _bn_affine(st, g_ref, be_ref):
    mean = st[0:1, :] * (1.0 / N)
    ex2 = st[1:2, :] * (1.0 / N)
    var = jnp.maximum(ex2 - mean * mean, 0.0)
    a = g_ref[...] * lax.rsqrt(var + EPS_BN)
    b = be_ref[...] - mean * a
    return a, b


def _layer_body(h_ref, a_ref, w1_ref, bb1_ref, w2_ref, bb2_ref, g_ref, be_ref,
                out_ref, z2s, st):
    p = pl.program_id(0)
    b = pl.program_id(1)

    @pl.when(p == 0)
    def _():
        _phase0(b, h_ref, a_ref, w1_ref, bb1_ref, w2_ref, bb2_ref, z2s, st)

    @pl.when(p == 1)
    def _():
        sa, sb = _bn_affine(st, g_ref, be_ref)
        hn = sa * z2s[pl.ds(b * ROWS, ROWS), :] + sb
        out_ref[0] = hn[:, :HALF]
        out_ref[1] = hn[:, HALF:]


def _layer_last_body(h_ref, a_ref, w1_ref, bb1_ref, w2_ref, bb2_ref,
                     g_ref, be_ref, wp_ref, bbp_ref, out_ref, z2s, st):
    p = pl.program_id(0)
    b = pl.program_id(1)

    @pl.when(p == 0)
    def _():
        _phase0(b, h_ref, a_ref, w1_ref, bb1_ref, w2_ref, bb2_ref, z2s, st)

    @pl.when(p == 1)
    def _():
        sa, sb = _bn_affine(st, g_ref, be_ref)
        hn = sa * z2s[pl.ds(b * ROWS, ROWS), :] + sb
        out_ref[...] = (
            lax.dot_general(hn, wp_ref[...], (((1,), (0,)), ((), ())), **_DOT)
            + bbp_ref[...])


def _in_block(p, b):
    return (0, jnp.where(p == 0, b, GRID - 1), 0)


_COMMON_IN = [
    pl.BlockSpec((NC, ROWS, HALF), _in_block),
    pl.BlockSpec((NC, ROWS, HALF), _in_block),
    pl.BlockSpec((D, D), lambda p, b: (0, 0)),
    pl.BlockSpec((1, D), lambda p, b: (0, 0)),
    pl.BlockSpec((D, D), lambda p, b: (0, 0)),
    pl.BlockSpec((1, D), lambda p, b: (0, 0)),
    pl.BlockSpec((1, D), lambda p, b: (0, 0)),
    pl.BlockSpec((1, D), lambda p, b: (0, 0)),
]

_TC_SCRATCH = [
    pltpu.VMEM((N, D), jnp.float32),
    pltpu.VMEM((8, D), jnp.float32),
]

_layer = pl.pallas_call(
    _layer_body,
    grid=(2, GRID),
    in_specs=_COMMON_IN,
    out_specs=pl.BlockSpec(
        (NC, ROWS, HALF), lambda p, b: (0, jnp.where(p == 0, 0, b), 0)),
    out_shape=jax.ShapeDtypeStruct((NC, N, HALF), jnp.float32),
    scratch_shapes=_TC_SCRATCH,
    compiler_params=pltpu.CompilerParams(
        dimension_semantics=("arbitrary", "arbitrary")),
)

_layer_last = pl.pallas_call(
    _layer_last_body,
    grid=(2, GRID),
    in_specs=_COMMON_IN + [
        pl.BlockSpec((D, OUT), lambda p, b: (0, 0)),
        pl.BlockSpec((1, OUT), lambda p, b: (0, 0)),
    ],
    out_specs=pl.BlockSpec(
        (ROWS, OUT), lambda p, b: (jnp.where(p == 0, 0, b), 0)),
    out_shape=jax.ShapeDtypeStruct((N, OUT), jnp.float32),
    scratch_shapes=_TC_SCRATCH,
    compiler_params=pltpu.CompilerParams(
        dimension_semantics=("arbitrary", "arbitrary")),
)


def kernel(x, edge_index, batch, W1, b1, W2, b2, gamma, beta, Wp, bp):
    del batch
    src = edge_index[0]
    dst = edge_index[1]

    npad = E_PAD - E
    pad_ids = jnp.arange(npad, dtype=jnp.int32)
    src_p = jnp.concatenate([src, pad_ids % N])
    dst_p = jnp.concatenate([dst, N + pad_ids % (ACC_ROWS - N)])
    src2 = jnp.stack([src_p, src_p + N]).reshape(NC, NS, NCHUNK, CHUNK)
    dst_r = dst_p.reshape(NS, NCHUNK, CHUNK)

    h2 = jnp.stack([x[:, :HALF], x[:, HALF:]])
    for l in range(L):
        agg = _sc_aggregate(h2.reshape(NC * N, HALF), src2, dst_r)
        args = (h2, agg, W1[l].T, b1[l].reshape(1, D), W2[l].T,
                b2[l].reshape(1, D), gamma[l].reshape(1, D),
                beta[l].reshape(1, D))
        if l < L - 1:
            h2 = _layer(*args)
        else:
            return _layer_last(*args, Wp.T, bp.reshape(1, OUT))


# speedup vs baseline: 1.1211x; 1.0083x over previous
import jax
import jax.numpy as jnp
from jax import lax
from jax.experimental import pallas as pl
from jax.experimental.pallas import tpu as pltpu, tpu_sc as plsc

N = 10000
E = 160000
D = 256
HALF = 128
OUT = 128
L = 3
EPS_BN = 1e-5

NC = 2
NS = 16
CHUNK = 128
NCHUNK = 80
EPT = CHUNK * NCHUNK
E_PAD = EPT * NS
ACC_ROWS = 10240
ROWS_PER_TILE_ZERO = ACC_ROWS // NS
ROWS_PER_TILE_OUT = 632
N_OUT = ROWS_PER_TILE_OUT * NS
NBUF = 2
NPHASE = 2
PCHUNK = NCHUNK // NPHASE

_mesh = plsc.VectorSubcoreMesh(
    core_axis_name="c", subcore_axis_name="s", num_cores=NC, num_subcores=NS)

_SC_SCRATCH = [
    pltpu.VMEM((PCHUNK, CHUNK), jnp.int32),
    pltpu.VMEM((PCHUNK, CHUNK), jnp.int32),
    pltpu.VMEM((CHUNK, HALF), jnp.float32),
    pltpu.VMEM((CHUNK, HALF), jnp.float32),
    pltpu.VMEM_SHARED((ACC_ROWS, HALF), jnp.float32),
    pltpu.SemaphoreType.DMA,
    pltpu.SemaphoreType.DMA,
]


def _sc_body(h_hbm, src_hbm, dst_hbm, out_hbm,
             sidx, didx, b0, b1, acc, s0, s1):
    c = lax.axis_index("c")
    s = lax.axis_index("s")
    bufs = (b0, b1)
    sems = (s0, s1)

    @pl.loop(0, CHUNK)
    def _zero_row(r):
        for k in range(HALF // 16):
            b0[r, pl.ds(k * 16, 16)] = jnp.zeros((16,), jnp.float32)

    for k in range(ROWS_PER_TILE_ZERO // CHUNK):
        pltpu.sync_copy(b0, acc.at[pl.ds(s * ROWS_PER_TILE_ZERO + k * CHUNK, CHUNK)])
    plsc.subcore_barrier()

    for p in range(NPHASE):
        pltpu.sync_copy(src_hbm.at[c, s, pl.ds(p * PCHUNK, PCHUNK)], sidx)
        pltpu.sync_copy(dst_hbm.at[s, pl.ds(p * PCHUNK, PCHUNK)], didx)

        for b in range(NBUF):
            pltpu.async_copy(h_hbm.at[sidx.at[b]], bufs[b], sems[b])

        @pl.loop(0, PCHUNK // NBUF - 1)
        def _main(t):
            j0 = t * NBUF
            for b in range(NBUF):
                j = j0 + b
                pltpu.make_async_copy(h_hbm.at[sidx.at[j]], bufs[b], sems[b]).wait()
                pltpu.sync_copy(bufs[b], acc.at[didx.at[j]], add=True)
                pltpu.async_copy(h_hbm.at[sidx.at[j + NBUF]], bufs[b], sems[b])

        for b in range(NBUF):
            j = PCHUNK - NBUF + b
            pltpu.make_async_copy(h_hbm.at[sidx.at[j]], bufs[b], sems[b]).wait()
            pltpu.sync_copy(bufs[b], acc.at[didx.at[j]], add=True)

    plsc.subcore_barrier()
    pltpu.sync_copy(acc.at[pl.ds(s * ROWS_PER_TILE_OUT, ROWS_PER_TILE_OUT)],
                    out_hbm.at[c, pl.ds(s * ROWS_PER_TILE_OUT, ROWS_PER_TILE_OUT)])


_sc_aggregate = pl.kernel(
    _sc_body,
    out_type=jax.ShapeDtypeStruct((NC, N_OUT, HALF), jnp.float32),
    mesh=_mesh,
    scratch_types=_SC_SCRATCH,
)


ROWS = 5000
GRID = N // ROWS
_DOT = dict(preferred_element_type=jnp.float32)


def _mlp_block(h_ref, a_ref, w1_ref, bb1_ref, w2_ref, bb2_ref):
    z = jnp.concatenate([h_ref[0] + a_ref[0], h_ref[1] + a_ref[1]], axis=-1)
    z1 = jnp.maximum(
        lax.dot_general(z, w1_ref[...], (((1,), (0,)), ((), ())), **_DOT)
        + bb1_ref[...], 0.0)
    return jnp.maximum(
        lax.dot_general(z1, w2_ref[...], (((1,), (0,)), ((), ())), **_DOT)
        + bb2_ref[...], 0.0)


def _phase0(b, h_ref, a_ref, w1_ref, bb1_ref, w2_ref, bb2_ref, z2s, st):
    z2 = _mlp_block(h_ref, a_ref, w1_ref, bb1_ref, w2_ref, bb2_ref)
    z2s[pl.ds(b * ROWS, ROWS), :] = z2

    @pl.when(b == 0)
    def _():
        st[...] = jnp.zeros_like(st)

    st[0:1, :] = st[0:1, :] + jnp.sum(z2, axis=0, keepdims=True)
    st[1:2, :] = st[1:2, :] + jnp.sum(z2 * z2, axis=0, keepdims=True)


def _bn_affine(st, g_ref, be_ref):
    mean = st[0:1, :] * (1.0 / N)
    ex2 = st[1:2, :] * (1.0 / N)
    var = jnp.maximum(ex2 - mean * mean, 0.0)
    a = g_ref[...] * lax.rsqrt(var + EPS_BN)
    b = be_ref[...] - mean * a
    return a, b


def _layer_body(h_ref, a_ref, w1_ref, bb1_ref, w2_ref, bb2_ref, g_ref, be_ref,
                out_ref, z2s, st):
    p = pl.program_id(0)
    b = pl.program_id(1)

    @pl.when(p == 0)
    def _():
        _phase0(b, h_ref, a_ref, w1_ref, bb1_ref, w2_ref, bb2_ref, z2s, st)

    @pl.when(p == 1)
    def _():
        sa, sb = _bn_affine(st, g_ref, be_ref)
        hn = sa * z2s[pl.ds(b * ROWS, ROWS), :] + sb
        out_ref[0] = hn[:, :HALF]
        out_ref[1] = hn[:, HALF:]


def _layer_last_body(h_ref, a_ref, w1_ref, bb1_ref, w2_ref, bb2_ref,
                     g_ref, be_ref, wp_ref, bbp_ref, out_ref, z2s, st):
    p = pl.program_id(0)
    b = pl.program_id(1)

    @pl.when(p == 0)
    def _():
        _phase0(b, h_ref, a_ref, w1_ref, bb1_ref, w2_ref, bb2_ref, z2s, st)

    @pl.when(p == 1)
    def _():
        sa, sb = _bn_affine(st, g_ref, be_ref)
        hn = sa * z2s[pl.ds(b * ROWS, ROWS), :] + sb
        out_ref[...] = (
            lax.dot_general(hn, wp_ref[...], (((1,), (0,)), ((), ())), **_DOT)
            + bbp_ref[...])


def _in_block(p, b):
    return (0, jnp.where(p == 0, b, GRID - 1), 0)


_COMMON_IN = [
    pl.BlockSpec((NC, ROWS, HALF), _in_block),
    pl.BlockSpec((NC, ROWS, HALF), _in_block),
    pl.BlockSpec((D, D), lambda p, b: (0, 0)),
    pl.BlockSpec((1, D), lambda p, b: (0, 0)),
    pl.BlockSpec((D, D), lambda p, b: (0, 0)),
    pl.BlockSpec((1, D), lambda p, b: (0, 0)),
    pl.BlockSpec((1, D), lambda p, b: (0, 0)),
    pl.BlockSpec((1, D), lambda p, b: (0, 0)),
]

_TC_SCRATCH = [
    pltpu.VMEM((N, D), jnp.float32),
    pltpu.VMEM((8, D), jnp.float32),
]

_layer = pl.pallas_call(
    _layer_body,
    grid=(2, GRID),
    in_specs=_COMMON_IN,
    out_specs=pl.BlockSpec(
        (NC, ROWS, HALF), lambda p, b: (0, jnp.where(p == 0, 0, b), 0)),
    out_shape=jax.ShapeDtypeStruct((NC, N, HALF), jnp.float32),
    scratch_shapes=_TC_SCRATCH,
    compiler_params=pltpu.CompilerParams(
        dimension_semantics=("arbitrary", "arbitrary")),
)

_layer_last = pl.pallas_call(
    _layer_last_body,
    grid=(2, GRID),
    in_specs=_COMMON_IN + [
        pl.BlockSpec((D, OUT), lambda p, b: (0, 0)),
        pl.BlockSpec((1, OUT), lambda p, b: (0, 0)),
    ],
    out_specs=pl.BlockSpec(
        (ROWS, OUT), lambda p, b: (jnp.where(p == 0, 0, b), 0)),
    out_shape=jax.ShapeDtypeStruct((N, OUT), jnp.float32),
    scratch_shapes=_TC_SCRATCH,
    compiler_params=pltpu.CompilerParams(
        dimension_semantics=("arbitrary", "arbitrary")),
)


def kernel(x, edge_index, batch, W1, b1, W2, b2, gamma, beta, Wp, bp):
    del batch
    src = edge_index[0]
    dst = edge_index[1]

    npad = E_PAD - E
    pad_ids = jnp.arange(npad, dtype=jnp.int32)
    src_p = jnp.concatenate([src, pad_ids % N])
    dst_p = jnp.concatenate([dst, N + pad_ids % (ACC_ROWS - N)])
    src2 = jnp.stack([src_p, src_p + N]).reshape(NC, NS, NCHUNK, CHUNK)
    dst_r = dst_p.reshape(NS, NCHUNK, CHUNK)

    h2 = jnp.stack([x[:, :HALF], x[:, HALF:]])
    for l in range(L):
        agg = _sc_aggregate(h2.reshape(NC * N, HALF), src2, dst_r)
        args = (h2, agg, W1[l].T, b1[l].reshape(1, D), W2[l].T,
                b2[l].reshape(1, D), gamma[l].reshape(1, D),
                beta[l].reshape(1, D))
        if l < L - 1:
            h2 = _layer(*args)
        else:
            return _layer_last(*args, Wp.T, bp.reshape(1, OUT))
